# baseline (speedup 1.0000x reference)
import functools

import jax
import jax.numpy as jnp
from jax import lax
from jax.experimental import pallas as pl
from jax.experimental.pallas import tpu as pltpu
from jax.experimental.pallas import tpu_sc as plsc

NUM_CORES = 2
NUM_SUBCORES = 16
NUM_WORKERS = NUM_CORES * NUM_SUBCORES
LANES = 16
NB = 128


@functools.partial(jax.jit, static_argnames=("hist", "d"))
def _gather_rows(vidx, csel, wv, *, hist, d):
    batch = vidx.shape[1]
    mesh = plsc.VectorSubcoreMesh(core_axis_name="c", subcore_axis_name="s")

    @functools.partial(
        pl.kernel,
        out_type=jax.ShapeDtypeStruct((hist, d, batch), jnp.float32),
        mesh=mesh,
        scratch_types=[
            pltpu.VMEM((hist, NB), jnp.int32),
            pltpu.VMEM((hist, NB), jnp.int32),
            [pltpu.VMEM((NB, 2 * d), jnp.float32) for _ in range(4)],
            [pltpu.VMEM((d, NB), jnp.float32) for _ in range(2)],
            [pltpu.SemaphoreType.DMA for _ in range(4)],
            [pltpu.SemaphoreType.DMA for _ in range(2)],
        ],
        compiler_params=pltpu.CompilerParams(
            use_tc_tiling_on_sc=True, needs_layout_passes=False
        ),
    )
    def body(vidx_hbm, csel_hbm, wv_hbm, out_hbm, iv, cv, gbufs, obufs, gsems, ssems):
        cid = lax.axis_index("c")
        sid = lax.axis_index("s")
        wid = sid * NUM_CORES + cid
        n0 = wid * NB
        pltpu.sync_copy(vidx_hbm.at[:, pl.ds(n0, NB)], iv)
        pltpu.sync_copy(csel_hbm.at[:, pl.ds(n0, NB)], cv)

        iota = lax.iota(jnp.int32, LANES)

        def gfire(h, b):
            pltpu.async_copy(wv_hbm.at[iv.at[h]], gbufs[b], gsems[b])

        def gwait(h, b):
            pltpu.make_async_copy(wv_hbm.at[iv.at[h]], gbufs[b], gsems[b]).wait()

        def sfire(h, b):
            pltpu.async_copy(obufs[b], out_hbm.at[h, :, pl.ds(n0, NB)], ssems[b])

        def swait(h, b):
            pltpu.make_async_copy(
                obufs[b], out_hbm.at[h, :, pl.ds(n0, NB)], ssems[b]
            ).wait()

        def compact(h, b, ob):
            csels = tuple(cv[h, pl.ds(nl0, LANES)] for nl0 in range(0, NB, LANES))

            def col(c, carry):
                for g in range(NB // LANES):
                    vals = plsc.load_gather(gbufs[b], [iota + g * LANES, carry[g] + c])
                    obufs[ob][c, pl.ds(g * LANES, LANES)] = vals
                return carry

            lax.fori_loop(0, d, col, csels)

        for b in range(4):
            gfire(b, b)

        def step(h, b, ob):
            @pl.when(h >= 2)
            def _():
                swait(h - 2, ob)

            gwait(h, b)
            compact(h, b, ob)
            sfire(h, ob)

            @pl.when(h + 4 < hist)
            def _():
                gfire(h + 4, b)

        def group(g, carry):
            for b in range(4):
                h = 4 * g + b
                step(h, b, b % 2)
            return carry

        lax.fori_loop(0, hist // 4, group, 0)
        for h in range(hist - hist % 4, hist):
            step(h, h % 4, h % 2)

        swait(hist - 2, (hist - 2) % 2)
        swait(hist - 1, (hist - 1) % 2)

    return body(vidx, csel, wv)


def kernel(input, weight):
    batch, hist = input.shape
    v, d = weight.shape
    assert batch == NUM_WORKERS * NB and hist % 2 == 0
    idxT = input.T
    vidx = idxT >> 1
    csel = (idxT & 1) * d
    wv = weight.reshape(v // 2, 2 * d)
    out3 = _gather_rows(vidx, csel, wv, hist=hist, d=d)
    return jnp.transpose(out3, (2, 0, 1))


# speedup vs baseline: 1.0015x; 1.0015x over previous
import functools

import jax
import jax.numpy as jnp
from jax import lax
from jax.experimental import pallas as pl
from jax.experimental.pallas import tpu as pltpu
from jax.experimental.pallas import tpu_sc as plsc

NUM_CORES = 2
NUM_SUBCORES = 16
NUM_WORKERS = NUM_CORES * NUM_SUBCORES
LANES = 16
NB = 128
_ENABLE_COMPACT = True


@functools.partial(jax.jit, static_argnames=("hist", "d"))
def _gather_rows(vidx, csel, wv, *, hist, d):
    batch = vidx.shape[1]
    mesh = plsc.VectorSubcoreMesh(core_axis_name="c", subcore_axis_name="s")

    @functools.partial(
        pl.kernel,
        out_type=jax.ShapeDtypeStruct((hist, d, batch), jnp.float32),
        mesh=mesh,
        scratch_types=[
            pltpu.VMEM((hist, NB), jnp.int32),
            pltpu.VMEM((hist, NB), jnp.int32),
            [pltpu.VMEM((NB, 2 * d), jnp.float32) for _ in range(4)],
            [pltpu.VMEM((d, NB), jnp.float32) for _ in range(2)],
            [pltpu.SemaphoreType.DMA for _ in range(4)],
            [pltpu.SemaphoreType.DMA for _ in range(2)],
        ],
        compiler_params=pltpu.CompilerParams(
            use_tc_tiling_on_sc=True, needs_layout_passes=False
        ),
    )
    def body(vidx_hbm, csel_hbm, wv_hbm, out_hbm, iv, cv, gbufs, obufs, gsems, ssems):
        cid = lax.axis_index("c")
        sid = lax.axis_index("s")
        wid = sid * NUM_CORES + cid
        n0 = wid * NB
        pltpu.sync_copy(vidx_hbm.at[:, pl.ds(n0, NB)], iv)
        pltpu.sync_copy(csel_hbm.at[:, pl.ds(n0, NB)], cv)

        iota = lax.iota(jnp.int32, LANES)

        def gfire(h, b):
            pltpu.async_copy(wv_hbm.at[iv.at[h]], gbufs[b], gsems[b])

        def gwait(h, b):
            pltpu.make_async_copy(wv_hbm.at[iv.at[h]], gbufs[b], gsems[b]).wait()

        def sfire(h, b):
            pltpu.async_copy(obufs[b], out_hbm.at[h, :, pl.ds(n0, NB)], ssems[b])

        def swait(h, b):
            pltpu.make_async_copy(
                obufs[b], out_hbm.at[h, :, pl.ds(n0, NB)], ssems[b]
            ).wait()

        nlvecs = tuple(iota + nl0 for nl0 in range(0, NB, LANES))

        def compact(h, b, ob):
            csels = tuple(cv[h, pl.ds(nl0, LANES)] for nl0 in range(0, NB, LANES))

            def col(c, carry):
                for g in range(NB // LANES):
                    vals = plsc.load_gather(gbufs[b], [nlvecs[g], csels[g] + c])
                    obufs[ob][c, pl.ds(g * LANES, LANES)] = vals
                return carry

            if _ENABLE_COMPACT:
                lax.fori_loop(0, d, col, 0)

        for b in range(4):
            gfire(b, b)

        def step(h, b, ob):
            @pl.when(h >= 2)
            def _():
                swait(h - 2, ob)

            gwait(h, b)
            compact(h, b, ob)
            sfire(h, ob)

            @pl.when(h + 4 < hist)
            def _():
                gfire(h + 4, b)

        def group(g, carry):
            for b in range(4):
                h = 4 * g + b
                step(h, b, b % 2)
            return carry

        lax.fori_loop(0, hist // 4, group, 0)
        for h in range(hist - hist % 4, hist):
            step(h, h % 4, h % 2)

        swait(hist - 2, (hist - 2) % 2)
        swait(hist - 1, (hist - 1) % 2)

    return body(vidx, csel, wv)


def kernel(input, weight):
    batch, hist = input.shape
    v, d = weight.shape
    assert batch == NUM_WORKERS * NB and hist % 2 == 0
    idxT = input.T
    vidx = idxT >> 1
    csel = (idxT & 1) * d
    wv = weight.reshape(v // 2, 2 * d)
    out3 = _gather_rows(vidx, csel, wv, hist=hist, d=d)
    return jnp.transpose(out3, (2, 0, 1))


# speedup vs baseline: 1.1847x; 1.1830x over previous
import functools

import jax
import jax.numpy as jnp
from jax import lax
from jax.experimental import pallas as pl
from jax.experimental.pallas import tpu as pltpu
from jax.experimental.pallas import tpu_sc as plsc

NUM_CORES = 2
NUM_SUBCORES = 16
NUM_WORKERS = NUM_CORES * NUM_SUBCORES
LANES = 16
NB = 128
_ENABLE_COMPACT = True


@functools.partial(jax.jit, static_argnames=("hist", "d"))
def _gather_rows(vidx, csel, wv, *, hist, d):
    batch = vidx.shape[1]
    mesh = plsc.VectorSubcoreMesh(core_axis_name="c", subcore_axis_name="s")

    @functools.partial(
        pl.kernel,
        out_type=jax.ShapeDtypeStruct((hist, d, batch), jnp.float32),
        mesh=mesh,
        scratch_types=[
            pltpu.VMEM((hist, NB), jnp.int32),
            pltpu.VMEM((hist, NB), jnp.int32),
            [pltpu.VMEM((NB, 2 * d), jnp.float32) for _ in range(4)],
            [pltpu.VMEM((d, NB), jnp.float32) for _ in range(2)],
            [pltpu.SemaphoreType.DMA for _ in range(4)],
            [pltpu.SemaphoreType.DMA for _ in range(2)],
        ],
        compiler_params=pltpu.CompilerParams(
            use_tc_tiling_on_sc=True, needs_layout_passes=False
        ),
    )
    def body(vidx_hbm, csel_hbm, wv_hbm, out_hbm, iv, cv, gbufs, obufs, gsems, ssems):
        cid = lax.axis_index("c")
        sid = lax.axis_index("s")
        wid = sid * NUM_CORES + cid
        n0 = wid * NB
        pltpu.sync_copy(vidx_hbm.at[:, pl.ds(n0, NB)], iv)
        pltpu.sync_copy(csel_hbm.at[:, pl.ds(n0, NB)], cv)

        iota = lax.iota(jnp.int32, LANES)

        def gfire(h, b):
            pltpu.async_copy(wv_hbm.at[iv.at[h]], gbufs[b], gsems[b])

        def gwait(h, b):
            pltpu.make_async_copy(wv_hbm.at[iv.at[h]], gbufs[b], gsems[b]).wait()

        def sfire(h, b):
            pltpu.async_copy(obufs[b], out_hbm.at[h, :, pl.ds(n0, NB)], ssems[b])

        def swait(h, b):
            pltpu.make_async_copy(
                obufs[b], out_hbm.at[h, :, pl.ds(n0, NB)], ssems[b]
            ).wait()

        nlvecs = tuple(iota + nl0 for nl0 in range(0, NB, LANES))

        def compact(h, b, ob):
            csels = tuple(cv[h, pl.ds(nl0, LANES)] for nl0 in range(0, NB, LANES))

            if _ENABLE_COMPACT:

                @plsc.parallel_loop(0, d, unroll=4)
                def col(c):
                    for g in range(NB // LANES):
                        vals = plsc.load_gather(gbufs[b], [nlvecs[g], csels[g] + c])
                        obufs[ob][c, pl.ds(g * LANES, LANES)] = vals

        for b in range(4):
            gfire(b, b)

        def step(h, b, ob):
            @pl.when(h >= 2)
            def _():
                swait(h - 2, ob)

            gwait(h, b)
            compact(h, b, ob)
            sfire(h, ob)

            @pl.when(h + 4 < hist)
            def _():
                gfire(h + 4, b)

        def group(g, carry):
            for b in range(4):
                h = 4 * g + b
                step(h, b, b % 2)
            return carry

        lax.fori_loop(0, hist // 4, group, 0)
        for h in range(hist - hist % 4, hist):
            step(h, h % 4, h % 2)

        swait(hist - 2, (hist - 2) % 2)
        swait(hist - 1, (hist - 1) % 2)

    return body(vidx, csel, wv)


def kernel(input, weight):
    batch, hist = input.shape
    v, d = weight.shape
    assert batch == NUM_WORKERS * NB and hist % 2 == 0
    idxT = input.T
    vidx = idxT >> 1
    csel = (idxT & 1) * d
    wv = weight.reshape(v // 2, 2 * d)
    out3 = _gather_rows(vidx, csel, wv, hist=hist, d=d)
    return jnp.transpose(out3, (2, 0, 1))


# speedup vs baseline: 1.1867x; 1.0016x over previous
import functools

import jax
import jax.numpy as jnp
from jax import lax
from jax.experimental import pallas as pl
from jax.experimental.pallas import tpu as pltpu
from jax.experimental.pallas import tpu_sc as plsc

NUM_CORES = 2
NUM_SUBCORES = 16
NUM_WORKERS = NUM_CORES * NUM_SUBCORES
LANES = 16
NB = 128
_ENABLE_COMPACT = True


@functools.partial(jax.jit, static_argnames=("hist", "d"))
def _gather_rows(vidx, csel, wv, *, hist, d):
    batch = vidx.shape[1]
    mesh = plsc.VectorSubcoreMesh(core_axis_name="c", subcore_axis_name="s")

    @functools.partial(
        pl.kernel,
        out_type=jax.ShapeDtypeStruct((hist, d, batch), jnp.float32),
        mesh=mesh,
        scratch_types=[
            pltpu.VMEM((hist, NB), jnp.int32),
            pltpu.VMEM((hist, NB), jnp.int32),
            [pltpu.VMEM((NB, 2 * d), jnp.float32) for _ in range(4)],
            [pltpu.VMEM((d, NB), jnp.float32) for _ in range(2)],
            [pltpu.SemaphoreType.DMA for _ in range(4)],
            [pltpu.SemaphoreType.DMA for _ in range(2)],
        ],
        compiler_params=pltpu.CompilerParams(
            use_tc_tiling_on_sc=True, needs_layout_passes=False
        ),
    )
    def body(vidx_hbm, csel_hbm, wv_hbm, out_hbm, iv, cv, gbufs, obufs, gsems, ssems):
        cid = lax.axis_index("c")
        sid = lax.axis_index("s")
        wid = sid * NUM_CORES + cid
        n0 = wid * NB
        pltpu.sync_copy(vidx_hbm.at[:, pl.ds(n0, NB)], iv)
        pltpu.sync_copy(csel_hbm.at[:, pl.ds(n0, NB)], cv)

        iota = lax.iota(jnp.int32, LANES)

        def gfire(h, b):
            pltpu.async_copy(wv_hbm.at[iv.at[h]], gbufs[b], gsems[b])

        def gwait(h, b):
            pltpu.make_async_copy(wv_hbm.at[iv.at[h]], gbufs[b], gsems[b]).wait()

        def sfire(h, b):
            pltpu.async_copy(obufs[b], out_hbm.at[h, :, pl.ds(n0, NB)], ssems[b])

        def swait(h, b):
            pltpu.make_async_copy(
                obufs[b], out_hbm.at[h, :, pl.ds(n0, NB)], ssems[b]
            ).wait()

        nlvecs = tuple(iota + nl0 for nl0 in range(0, NB, LANES))

        def compact(h, b, ob):
            csels = tuple(cv[h, pl.ds(nl0, LANES)] for nl0 in range(0, NB, LANES))

            if _ENABLE_COMPACT:

                @plsc.parallel_loop(0, d, unroll=8)
                def col(c):
                    for g in range(NB // LANES):
                        vals = plsc.load_gather(gbufs[b], [nlvecs[g], csels[g] + c])
                        obufs[ob][c, pl.ds(g * LANES, LANES)] = vals

        for b in range(4):
            gfire(b, b)

        def step(h, b, ob):
            @pl.when(h >= 2)
            def _():
                swait(h - 2, ob)

            gwait(h, b)
            compact(h, b, ob)
            sfire(h, ob)

            @pl.when(h + 4 < hist)
            def _():
                gfire(h + 4, b)

        def group(g, carry):
            for b in range(4):
                h = 4 * g + b
                step(h, b, b % 2)
            return carry

        lax.fori_loop(0, hist // 4, group, 0)
        for h in range(hist - hist % 4, hist):
            step(h, h % 4, h % 2)

        swait(hist - 2, (hist - 2) % 2)
        swait(hist - 1, (hist - 1) % 2)

    return body(vidx, csel, wv)


def kernel(input, weight):
    batch, hist = input.shape
    v, d = weight.shape
    assert batch == NUM_WORKERS * NB and hist % 2 == 0
    idxT = input.T
    vidx = idxT >> 1
    csel = (idxT & 1) * d
    wv = weight.reshape(v // 2, 2 * d)
    out3 = _gather_rows(vidx, csel, wv, hist=hist, d=d)
    return jnp.transpose(out3, (2, 0, 1))
